# baseline (speedup 1.0000x reference)
import jax
import jax.numpy as jnp
from jax import lax
from jax.experimental import pallas as pl
from jax.experimental.pallas import tpu as pltpu

_IOU_T = 0.3
_ALPHA = 0.25
_SUB = 128
_K = 4


def _step(nb, cls_ref, anc_ref, reg_ref, ann_ref,
          cls_out, reg_out, np_out, key_ref):
    k = pl.program_id(1)
    pcol = (nb + _SUB - 1) // _SUB
    tail = nb - _SUB * (pcol - 1)

    @pl.when(k == 0)
    def _assign():
        ax1 = anc_ref[0]
        ay1 = anc_ref[1]
        ax2 = anc_ref[2]
        ay2 = anc_ref[3]
        shp = ax1.shape

        area_a = (ax2 - ax1) * (ay2 - ay1)
        best = jnp.full(shp, -1.0, jnp.float32)
        gx1 = jnp.zeros(shp, jnp.float32)
        gy1 = jnp.zeros(shp, jnp.float32)
        gx2 = jnp.zeros(shp, jnp.float32)
        gy2 = jnp.zeros(shp, jnp.float32)
        glab = jnp.zeros(shp, jnp.float32)
        m = ann_ref.shape[1]
        for j in range(m):
            bx1 = ann_ref[0, j, 0]
            by1 = ann_ref[0, j, 1]
            bx2 = ann_ref[0, j, 2]
            by2 = ann_ref[0, j, 3]
            blab = ann_ref[0, j, 4]
            iw = jnp.maximum(
                jnp.minimum(ax2, bx2) - jnp.maximum(ax1, bx1), 0.0)
            ih = jnp.maximum(
                jnp.minimum(ay2, by2) - jnp.maximum(ay1, by1), 0.0)
            inter = iw * ih
            area_b = (bx2 - bx1) * (by2 - by1)
            ua = jnp.maximum(area_a + (area_b - inter), 1e-08)
            iou = inter / ua
            upd = iou > best
            best = jnp.maximum(best, iou)
            gx1 = jnp.where(upd, bx1, gx1)
            gy1 = jnp.where(upd, by1, gy1)
            gx2 = jnp.where(upd, bx2, gx2)
            gy2 = jnp.where(upd, by2, gy2)
            glab = jnp.where(upd, blab, glab)

        sub = lax.broadcasted_iota(jnp.int32, shp, 0)
        lane = lax.broadcasted_iota(jnp.int32, shp, 1)
        local = (lane % pcol) * _SUB + sub
        valid = local < nb

        positive = best >= _IOU_T + 0.1
        neg_row = jnp.logical_and(best < _IOU_T, valid)
        pos_f = positive.astype(jnp.float32)
        label = glab.astype(jnp.int32)
        key = jnp.where(positive, label,
                        jnp.where(neg_row, -1, -2)).astype(jnp.int32)
        for s in range(_K):
            key_ref[_SUB * s:_SUB * (s + 1), :] = key[:, pcol * s:
                                                      pcol * (s + 1)]

        aw = ax2 - ax1
        ah = ay2 - ay1
        acx = ax1 + 0.5 * aw
        acy = ay1 + 0.5 * ah
        aw_s = jnp.where(positive, aw, 1.0)
        ah_s = jnp.where(positive, ah, 1.0)
        gw = gx2 - gx1
        gh = gy2 - gy1
        gcx = gx1 + 0.5 * gw
        gcy = gy1 + 0.5 * gh
        gw = jnp.maximum(gw, 1.0)
        gh = jnp.maximum(gh, 1.0)
        tdx = (gcx - acx) / aw_s / 0.1
        tdy = (gcy - acy) / ah_s / 0.1
        tdw = jnp.log(gw / aw_s) / 0.2
        tdh = jnp.log(gh / ah_s) / 0.2

        rsum = jnp.float32(0.0)
        for t_i, t in enumerate((tdx, tdy, tdw, tdh)):
            d = jnp.abs(t - reg_ref[0, t_i])
            rl = jnp.where(d <= 1.0, 0.5 * d * d, d - 0.5)
            rsum = rsum + jnp.sum(rl * pos_f)
        reg_out[0, 0, 0] = rsum
        np_out[0, 0, 0] = jnp.sum(pos_f)
        cls_out[0, 0, 0] = 0.0

    c = cls_ref.shape[3]
    cl_iota = lax.broadcasted_iota(jnp.int32, (_SUB, c), 1)
    base = _SUB * k
    acc = jnp.zeros((8, c), jnp.float32)
    for j in range(pcol):
        rows = _SUB if j < pcol - 1 else tail
        ch = cls_ref[0, 0, _SUB * j:_SUB * j + rows, :]
        ch = jnp.clip(ch, 0.0001, 1.0 - 0.0001)
        kcol = key_ref[pl.ds(base, rows), j:j + 1]
        kb = jnp.broadcast_to(kcol, (rows, c))
        t1 = cl_iota[:rows] == kb
        larg = jnp.where(t1, ch, 1.0 - ch)
        larg = jnp.where(kb == -2, 1.0, larg)
        pfac = 1.0 - larg
        w = jnp.where(t1, -_ALPHA, _ALPHA - 1.0)
        fl = (w * pfac) * (pfac * jnp.log(larg))
        while fl.shape[0] > 8 and fl.shape[0] % 2 == 0:
            h = fl.shape[0] // 2
            fl = fl[:h] + fl[h:]
        acc = acc + fl
    cls_out[0, 0, 0] = cls_out[0, 0, 0] + jnp.sum(acc)


def _fold(x, k, nb, pcol):
    lead = x.shape[:-2]
    x = x.reshape(lead + (k, nb, 4))
    x = jnp.pad(x, [(0, 0)] * len(lead) + [(0, 0), (0, pcol * _SUB - nb),
                                           (0, 0)])
    x = x.reshape(lead + (k, pcol, _SUB, 4))
    nl = len(lead)
    perm = tuple(range(nl)) + (nl + 3, nl + 2, nl, nl + 1)
    x = jnp.transpose(x, perm)
    return x.reshape(lead + (4, _SUB, k * pcol))


def kernel(classifications, regressions, anchors, annotations):
    b, n, c = classifications.shape
    nb = n // _K
    pcol = (nb + _SUB - 1) // _SUB

    anc4 = _fold(anchors[0], _K, nb, pcol)
    reg4 = _fold(regressions, _K, nb, pcol)

    sout = lambda: pl.BlockSpec((1, 1, 1), lambda bi, k: (bi, 0, 0),
                                memory_space=pltpu.SMEM)
    cls_sum, reg_sum, npos = pl.pallas_call(
        lambda *a: _step(nb, *a),
        grid=(b, _K),
        in_specs=[
            pl.BlockSpec((1, 1, nb, c), lambda bi, k: (bi, k, 0, 0)),
            pl.BlockSpec((4, _SUB, _K * pcol), lambda bi, k: (0, 0, 0)),
            pl.BlockSpec((1, 4, _SUB, _K * pcol),
                         lambda bi, k: (bi, 0, 0, 0)),
            pl.BlockSpec((1, 32, 5), lambda bi, k: (bi, 0, 0),
                         memory_space=pltpu.SMEM),
        ],
        out_specs=[sout(), sout(), sout()],
        out_shape=[jax.ShapeDtypeStruct((b, 1, 1), jnp.float32)] * 3,
        scratch_shapes=[pltpu.VMEM((_K * _SUB, pcol), jnp.int32)],
    )(classifications.reshape(b, _K, nb, c), anc4, reg4, annotations)

    num_pos = npos[:, 0, 0]
    cls_losses = cls_sum[:, 0, 0] / jnp.clip(num_pos, 1.0, None)
    reg_losses = jnp.where(
        num_pos > 0,
        reg_sum[:, 0, 0] / jnp.clip(num_pos * 4.0, 1.0, None),
        0.0,
    )
    cls_out = jnp.mean(cls_losses, keepdims=True)
    reg_out = jnp.mean(reg_losses, keepdims=True)
    num_detected = jnp.sum(num_pos).astype(jnp.int32)
    return (cls_out, reg_out, num_detected)


# speedup vs baseline: 1.9832x; 1.9832x over previous
import jax
import jax.numpy as jnp
from jax import lax
from jax.experimental import pallas as pl
from jax.experimental.pallas import tpu as pltpu

_IOU_T = 0.3
_ALPHA = 0.25
_SUB = 128


def _fused_block(n_valid, cls_ref, anc_ref, reg_ref, ann_ref,
                 cls_out, reg_out, np_out):
    ax1 = anc_ref[0]
    ay1 = anc_ref[1]
    ax2 = anc_ref[2]
    ay2 = anc_ref[3]
    shp = ax1.shape

    area_a = (ax2 - ax1) * (ay2 - ay1)
    best = jnp.full(shp, -1.0, jnp.float32)
    gx1 = jnp.zeros(shp, jnp.float32)
    gy1 = jnp.zeros(shp, jnp.float32)
    gx2 = jnp.zeros(shp, jnp.float32)
    gy2 = jnp.zeros(shp, jnp.float32)
    glab = jnp.zeros(shp, jnp.float32)
    m = ann_ref.shape[1]
    for j in range(m):
        bx1 = ann_ref[0, j, 0]
        by1 = ann_ref[0, j, 1]
        bx2 = ann_ref[0, j, 2]
        by2 = ann_ref[0, j, 3]
        blab = ann_ref[0, j, 4]
        iw = jnp.maximum(jnp.minimum(ax2, bx2) - jnp.maximum(ax1, bx1), 0.0)
        ih = jnp.maximum(jnp.minimum(ay2, by2) - jnp.maximum(ay1, by1), 0.0)
        inter = iw * ih
        area_b = (bx2 - bx1) * (by2 - by1)
        ua = jnp.maximum(area_a + (area_b - inter), 1e-08)
        iou = inter / ua
        upd = iou > best
        best = jnp.maximum(best, iou)
        gx1 = jnp.where(upd, bx1, gx1)
        gy1 = jnp.where(upd, by1, gy1)
        gx2 = jnp.where(upd, bx2, gx2)
        gy2 = jnp.where(upd, by2, gy2)
        glab = jnp.where(upd, blab, glab)

    sub = lax.broadcasted_iota(jnp.int32, shp, 0)
    lane = lax.broadcasted_iota(jnp.int32, shp, 1)
    valid = (lane * _SUB + sub) < n_valid

    positive = best >= _IOU_T + 0.1
    neg_row = jnp.logical_and(best < _IOU_T, valid)
    pos_f = positive.astype(jnp.float32)
    label = glab.astype(jnp.int32)
    key = jnp.where(positive, label,
                    jnp.where(neg_row, -1, -2)).astype(jnp.int32)

    aw = ax2 - ax1
    ah = ay2 - ay1
    acx = ax1 + 0.5 * aw
    acy = ay1 + 0.5 * ah
    aw_s = jnp.where(positive, aw, 1.0)
    ah_s = jnp.where(positive, ah, 1.0)
    gw = gx2 - gx1
    gh = gy2 - gy1
    gcx = gx1 + 0.5 * gw
    gcy = gy1 + 0.5 * gh
    gw = jnp.maximum(gw, 1.0)
    gh = jnp.maximum(gh, 1.0)
    tdx = (gcx - acx) / aw_s / 0.1
    tdy = (gcy - acy) / ah_s / 0.1
    tdw = jnp.log(gw / aw_s) / 0.2
    tdh = jnp.log(gh / ah_s) / 0.2

    rsum = jnp.float32(0.0)
    for k, t in enumerate((tdx, tdy, tdw, tdh)):
        d = jnp.abs(t - reg_ref[0, k])
        rl = jnp.where(d <= 1.0, 0.5 * d * d, d - 0.5)
        rsum = rsum + jnp.sum(rl * pos_f)
    reg_out[0, 0, 0] = rsum
    np_out[0, 0, 0] = jnp.sum(pos_f)

    n, c = cls_ref.shape[1], cls_ref.shape[2]
    cl_iota = lax.broadcasted_iota(jnp.int32, (_SUB, c), 1)
    acc = jnp.zeros((8, c), jnp.float32)
    g = 0
    row = 0
    while row < n:
        rows = min(_SUB, n - row)
        ch = cls_ref[0, row:row + rows, :]
        ch = jnp.clip(ch, 0.0001, 1.0 - 0.0001)
        kb = jnp.broadcast_to(key[:rows, g:g + 1], (rows, c))
        t1 = cl_iota[:rows] == kb
        larg = jnp.where(t1, ch, 1.0 - ch)
        larg = jnp.where(kb == -2, 1.0, larg)
        pfac = 1.0 - larg
        w = jnp.where(t1, -_ALPHA, _ALPHA - 1.0)
        fl = (w * pfac) * (pfac * jnp.log(larg))
        while fl.shape[0] > 8 and fl.shape[0] % 2 == 0:
            h = fl.shape[0] // 2
            fl = fl[:h] + fl[h:]
        acc = acc + fl
        row += rows
        g += 1
    cls_out[0, 0, 0] = jnp.sum(acc)


def kernel(classifications, regressions, anchors, annotations):
    b, n, c = classifications.shape
    g = (n + _SUB - 1) // _SUB
    n_pad = g * _SUB

    anc4 = jnp.pad(anchors[0], ((0, n_pad - n), (0, 0)))
    anc4 = jnp.transpose(anc4, (1, 0)).reshape(4, g, _SUB)
    anc4 = jnp.transpose(anc4, (0, 2, 1))
    reg4 = jnp.pad(regressions, ((0, 0), (0, n_pad - n), (0, 0)))
    reg4 = jnp.transpose(reg4, (0, 2, 1)).reshape(b, 4, g, _SUB)
    reg4 = jnp.transpose(reg4, (0, 1, 3, 2))

    sout = lambda: pl.BlockSpec((1, 1, 1), lambda bi: (bi, 0, 0),
                                memory_space=pltpu.SMEM)
    cls_sum, reg_sum, npos = pl.pallas_call(
        lambda *a: _fused_block(n, *a),
        grid=(b,),
        in_specs=[
            pl.BlockSpec((1, n, c), lambda bi: (bi, 0, 0)),
            pl.BlockSpec((4, _SUB, g), lambda bi: (0, 0, 0)),
            pl.BlockSpec((1, 4, _SUB, g), lambda bi: (bi, 0, 0, 0)),
            pl.BlockSpec((1, 32, 5), lambda bi: (bi, 0, 0),
                         memory_space=pltpu.SMEM),
        ],
        out_specs=[sout(), sout(), sout()],
        out_shape=[jax.ShapeDtypeStruct((b, 1, 1), jnp.float32)] * 3,
    )(classifications, anc4, reg4, annotations)

    num_pos = npos[:, 0, 0]
    cls_losses = cls_sum[:, 0, 0] / jnp.clip(num_pos, 1.0, None)
    reg_losses = jnp.where(
        num_pos > 0,
        reg_sum[:, 0, 0] / jnp.clip(num_pos * 4.0, 1.0, None),
        0.0,
    )
    cls_out = jnp.mean(cls_losses, keepdims=True)
    reg_out = jnp.mean(reg_losses, keepdims=True)
    num_detected = jnp.sum(num_pos).astype(jnp.int32)
    return (cls_out, reg_out, num_detected)
